# Initial kernel scaffold; baseline (speedup 1.0000x reference)
import functools

import jax
import jax.numpy as jnp
from jax import lax
from jax.experimental import pallas as pl
from jax.experimental.pallas import tpu as pltpu

HIDDEN = 768
NUM_EXPERTS = 64
EXPERT_DIM = 512
SEQ = 2048

TILE_T = 64
TILES = SEQ // TILE_T + NUM_EXPERTS
ROWS = TILES * TILE_T

_INTERPRET = False


def _router_body(x_ref, wr_ref, probs_ref, pos_ref, wgt_ref, be_ref):
    x = x_ref[...]
    wr = wr_ref[...]
    logits = lax.dot_general(x, wr, (((1,), (1,)), ((), ())),
                             preferred_element_type=jnp.float32)
    m = jnp.max(logits, axis=1, keepdims=True)
    ex = jnp.exp(logits - m)
    probs = ex / jnp.sum(ex, axis=1, keepdims=True)
    probs_ref[...] = probs

    pmax = jnp.max(probs, axis=1, keepdims=True)
    col = lax.broadcasted_iota(jnp.int32, (SEQ, NUM_EXPERTS), 1)
    eidx = jnp.min(jnp.where(probs >= pmax, col, NUM_EXPERTS), axis=1,
                   keepdims=True)
    wgt_ref[...] = pmax / (pmax + 1e-9)

    onehot = (col == eidx).astype(jnp.float32)

    ones_row = jnp.ones((8, SEQ), dtype=jnp.float32)
    counts = lax.dot_general(ones_row, onehot, (((1,), (0,)), ((), ())),
                             preferred_element_type=jnp.float32)[0:1]
    ptc = jnp.floor((counts + (TILE_T - 1)) * (1.0 / TILE_T))
    er = lax.broadcasted_iota(jnp.int32, (NUM_EXPERTS, NUM_EXPERTS), 0)
    ec = lax.broadcasted_iota(jnp.int32, (NUM_EXPERTS, NUM_EXPERTS), 1)
    upper = (er <= ec).astype(jnp.float32)
    tile_start = lax.dot_general(ptc, upper, (((1,), (0,)), ((), ())),
                                 preferred_element_type=jnp.float32) - ptc
    row_start = tile_start * float(TILE_T)

    tr = lax.broadcasted_iota(jnp.int32, (SEQ, SEQ), 0)
    tc = lax.broadcasted_iota(jnp.int32, (SEQ, SEQ), 1)
    lower = (tc <= tr).astype(jnp.float32)
    csum = lax.dot_general(lower, onehot, (((1,), (0,)), ((), ())),
                           preferred_element_type=jnp.float32)
    posmat = onehot * (csum - 1.0 + row_start)
    pos = jnp.sum(posmat, axis=1, keepdims=True)
    pos_ref[...] = pos.astype(jnp.int32)

    ti = lax.broadcasted_iota(jnp.float32, (TILES, NUM_EXPERTS), 0)
    le = (jnp.broadcast_to(tile_start, (TILES, NUM_EXPERTS)) <= ti)
    be = jnp.sum(le.astype(jnp.float32), axis=1, keepdims=True) - 1.0
    be_ref[...] = be.astype(jnp.int32)


def _run_router(x_flat, W_router):
    return pl.pallas_call(
        _router_body,
        out_shape=(
            jax.ShapeDtypeStruct((SEQ, NUM_EXPERTS), jnp.float32),
            jax.ShapeDtypeStruct((SEQ, 1), jnp.int32),
            jax.ShapeDtypeStruct((SEQ, 1), jnp.float32),
            jax.ShapeDtypeStruct((TILES, 1), jnp.int32),
        ),
        interpret=_INTERPRET,
    )(x_flat, W_router)


def _group_ffn_body(be_ref, x_ref, wu_ref, wd_ref, o_ref):
    xb = x_ref[...]
    wu = wu_ref[0]
    wd = wd_ref[0]
    y = lax.dot_general(xb, wu, (((1,), (1,)), ((), ())),
                        preferred_element_type=jnp.float32)
    gate = y[:, :EXPERT_DIM]
    up = y[:, EXPERT_DIM:]
    h = gate * jax.nn.sigmoid(gate) * up
    o_ref[...] = lax.dot_general(h, wd, (((1,), (1,)), ((), ())),
                                 preferred_element_type=jnp.float32)


def _run_group_ffn(be, xs, W_up, W_down):
    grid_spec = pltpu.PrefetchScalarGridSpec(
        num_scalar_prefetch=1,
        grid=(TILES,),
        in_specs=[
            pl.BlockSpec((TILE_T, HIDDEN), lambda i, be_s: (i, 0)),
            pl.BlockSpec((1, 2 * EXPERT_DIM, HIDDEN),
                         lambda i, be_s: (be_s[i], 0, 0)),
            pl.BlockSpec((1, HIDDEN, EXPERT_DIM),
                         lambda i, be_s: (be_s[i], 0, 0)),
        ],
        out_specs=pl.BlockSpec((TILE_T, HIDDEN), lambda i, be_s: (i, 0)),
    )
    return pl.pallas_call(
        _group_ffn_body,
        grid_spec=grid_spec,
        out_shape=jax.ShapeDtypeStruct((ROWS, HIDDEN), jnp.float32),
        interpret=_INTERPRET,
    )(be, xs, W_up, W_down)


def _combine_body(x_ref, yg_ref, wgt_ref, wu_ref, wd_ref, o_ref):
    x = x_ref[...]
    wu = wu_ref[...]
    wd = wd_ref[...]
    y = lax.dot_general(x, wu, (((1,), (1,)), ((), ())),
                        preferred_element_type=jnp.float32)
    gate = y[:, :EXPERT_DIM]
    up = y[:, EXPERT_DIM:]
    h = gate * jax.nn.sigmoid(gate) * up
    shared = lax.dot_general(h, wd, (((1,), (1,)), ((), ())),
                             preferred_element_type=jnp.float32)
    o_ref[...] = shared + wgt_ref[...] * yg_ref[...]


def _run_combine(x_flat, yg, wgt, W_up_shared, W_down_shared):
    blk = 256
    grid_spec = pl.GridSpec(
        grid=(SEQ // blk,),
        in_specs=[
            pl.BlockSpec((blk, HIDDEN), lambda i: (i, 0)),
            pl.BlockSpec((blk, HIDDEN), lambda i: (i, 0)),
            pl.BlockSpec((blk, 1), lambda i: (i, 0)),
            pl.BlockSpec((2 * EXPERT_DIM, HIDDEN), lambda i: (0, 0)),
            pl.BlockSpec((HIDDEN, EXPERT_DIM), lambda i: (0, 0)),
        ],
        out_specs=pl.BlockSpec((blk, HIDDEN), lambda i: (i, 0)),
    )
    return pl.pallas_call(
        _combine_body,
        grid_spec=grid_spec,
        out_shape=jax.ShapeDtypeStruct((SEQ, HIDDEN), jnp.float32),
        interpret=_INTERPRET,
    )(x_flat, yg, wgt, W_up_shared, W_down_shared)


def _sc_scatter_rows(x_flat, pos1d):
    xs = jnp.zeros((ROWS, HIDDEN), jnp.float32)
    return xs.at[pos1d].set(x_flat)


def _sc_gather_rows(ys, pos1d):
    return ys[pos1d]


def kernel(x, W_router, W_up, W_down, W_up_shared, W_down_shared):
    bsz, seq, hidden = x.shape
    x_flat = x.reshape(-1, hidden)
    probs, pos, wgt, be = _run_router(x_flat, W_router)
    pos1d = pos.reshape(-1)
    be1d = be.reshape(-1)
    xs = _sc_scatter_rows(x_flat, pos1d)
    ys = _run_group_ffn(be1d, xs, W_up, W_down)
    yg = _sc_gather_rows(ys, pos1d)
    out = _run_combine(x_flat, yg, wgt, W_up_shared, W_down_shared)
    return out.reshape(bsz, seq, hidden), probs


# speedup vs baseline: 4.3301x; 4.3301x over previous
import functools

import jax
import jax.numpy as jnp
from jax import lax
from jax.experimental import pallas as pl
from jax.experimental.pallas import tpu as pltpu

HIDDEN = 768
NUM_EXPERTS = 64
EXPERT_DIM = 512
SEQ = 2048

TILE_T = 64
TILES = SEQ // TILE_T + NUM_EXPERTS
ROWS = TILES * TILE_T

_INTERPRET = False


def _router_body(x_ref, wr_ref, probs_ref, pos_ref, wgt_ref, be_ref):
    x = x_ref[...]
    wr = wr_ref[...]
    logits = lax.dot_general(x, wr, (((1,), (1,)), ((), ())),
                             preferred_element_type=jnp.float32)
    m = jnp.max(logits, axis=1, keepdims=True)
    ex = jnp.exp(logits - m)
    probs = ex / jnp.sum(ex, axis=1, keepdims=True)
    probs_ref[...] = probs

    pmax = jnp.max(probs, axis=1, keepdims=True)
    col = lax.broadcasted_iota(jnp.int32, (SEQ, NUM_EXPERTS), 1)
    eidx = jnp.min(jnp.where(probs >= pmax, col, NUM_EXPERTS), axis=1,
                   keepdims=True)
    wgt_ref[...] = pmax / (pmax + 1e-9)

    onehot = (col == eidx).astype(jnp.float32)

    ones_row = jnp.ones((8, SEQ), dtype=jnp.float32)
    counts = lax.dot_general(ones_row, onehot, (((1,), (0,)), ((), ())),
                             preferred_element_type=jnp.float32)[0:1]
    ptc = jnp.floor((counts + (TILE_T - 1)) * (1.0 / TILE_T))
    er = lax.broadcasted_iota(jnp.int32, (NUM_EXPERTS, NUM_EXPERTS), 0)
    ec = lax.broadcasted_iota(jnp.int32, (NUM_EXPERTS, NUM_EXPERTS), 1)
    upper = (er <= ec).astype(jnp.float32)
    tile_start = lax.dot_general(ptc, upper, (((1,), (0,)), ((), ())),
                                 preferred_element_type=jnp.float32) - ptc
    row_start = tile_start * float(TILE_T)

    tr = lax.broadcasted_iota(jnp.int32, (SEQ, SEQ), 0)
    tc = lax.broadcasted_iota(jnp.int32, (SEQ, SEQ), 1)
    lower = (tc <= tr).astype(jnp.float32)
    csum = lax.dot_general(lower, onehot, (((1,), (0,)), ((), ())),
                           preferred_element_type=jnp.float32)
    posmat = onehot * (csum - 1.0 + row_start)
    pos = jnp.sum(posmat, axis=1, keepdims=True)
    pos_ref[...] = pos.astype(jnp.int32)

    ti = lax.broadcasted_iota(jnp.int32, (TILES, NUM_EXPERTS), 0).astype(
        jnp.float32)
    le = (jnp.broadcast_to(tile_start, (TILES, NUM_EXPERTS)) <= ti)
    be = jnp.sum(le.astype(jnp.float32), axis=1, keepdims=True) - 1.0
    be_ref[...] = be.astype(jnp.int32)


def _run_router(x_flat, W_router):
    return pl.pallas_call(
        _router_body,
        out_shape=(
            jax.ShapeDtypeStruct((SEQ, NUM_EXPERTS), jnp.float32),
            jax.ShapeDtypeStruct((SEQ, 1), jnp.int32),
            jax.ShapeDtypeStruct((SEQ, 1), jnp.float32),
            jax.ShapeDtypeStruct((TILES, 1), jnp.int32),
        ),
        interpret=_INTERPRET,
    )(x_flat, W_router)


def _group_ffn_body(be_ref, x_ref, wu_ref, wd_ref, o_ref):
    xb = x_ref[...]
    wu = wu_ref[0]
    wd = wd_ref[0]
    y = lax.dot_general(xb, wu, (((1,), (1,)), ((), ())),
                        preferred_element_type=jnp.float32)
    gate = y[:, :EXPERT_DIM]
    up = y[:, EXPERT_DIM:]
    h = gate * jax.nn.sigmoid(gate) * up
    o_ref[...] = lax.dot_general(h, wd, (((1,), (1,)), ((), ())),
                                 preferred_element_type=jnp.float32)


def _run_group_ffn(be, xs, W_up, W_down):
    grid_spec = pltpu.PrefetchScalarGridSpec(
        num_scalar_prefetch=1,
        grid=(TILES,),
        in_specs=[
            pl.BlockSpec((TILE_T, HIDDEN), lambda i, be_s: (i, 0)),
            pl.BlockSpec((1, 2 * EXPERT_DIM, HIDDEN),
                         lambda i, be_s: (be_s[i], 0, 0)),
            pl.BlockSpec((1, HIDDEN, EXPERT_DIM),
                         lambda i, be_s: (be_s[i], 0, 0)),
        ],
        out_specs=pl.BlockSpec((TILE_T, HIDDEN), lambda i, be_s: (i, 0)),
    )
    return pl.pallas_call(
        _group_ffn_body,
        grid_spec=grid_spec,
        out_shape=jax.ShapeDtypeStruct((ROWS, HIDDEN), jnp.float32),
        interpret=_INTERPRET,
    )(be, xs, W_up, W_down)


def _combine_body(x_ref, yg_ref, wgt_ref, wu_ref, wd_ref, o_ref):
    x = x_ref[...]
    wu = wu_ref[...]
    wd = wd_ref[...]
    y = lax.dot_general(x, wu, (((1,), (1,)), ((), ())),
                        preferred_element_type=jnp.float32)
    gate = y[:, :EXPERT_DIM]
    up = y[:, EXPERT_DIM:]
    h = gate * jax.nn.sigmoid(gate) * up
    shared = lax.dot_general(h, wd, (((1,), (1,)), ((), ())),
                             preferred_element_type=jnp.float32)
    o_ref[...] = shared + wgt_ref[...] * yg_ref[...]


def _run_combine(x_flat, yg, wgt, W_up_shared, W_down_shared):
    blk = 256
    grid_spec = pl.GridSpec(
        grid=(SEQ // blk,),
        in_specs=[
            pl.BlockSpec((blk, HIDDEN), lambda i: (i, 0)),
            pl.BlockSpec((blk, HIDDEN), lambda i: (i, 0)),
            pl.BlockSpec((blk, 1), lambda i: (i, 0)),
            pl.BlockSpec((2 * EXPERT_DIM, HIDDEN), lambda i: (0, 0)),
            pl.BlockSpec((HIDDEN, EXPERT_DIM), lambda i: (0, 0)),
        ],
        out_specs=pl.BlockSpec((blk, HIDDEN), lambda i: (i, 0)),
    )
    return pl.pallas_call(
        _combine_body,
        grid_spec=grid_spec,
        out_shape=jax.ShapeDtypeStruct((SEQ, HIDDEN), jnp.float32),
        interpret=_INTERPRET,
    )(x_flat, yg, wgt, W_up_shared, W_down_shared)


def _sc_scatter_rows(x_flat, pos1d):
    xs = jnp.zeros((ROWS, HIDDEN), jnp.float32)
    return xs.at[pos1d].set(x_flat)


def _sc_gather_rows(ys, pos1d):
    return ys[pos1d]


def kernel(x, W_router, W_up, W_down, W_up_shared, W_down_shared):
    bsz, seq, hidden = x.shape
    x_flat = x.reshape(-1, hidden)
    probs, pos, wgt, be = _run_router(x_flat, W_router)
    pos1d = pos.reshape(-1)
    be1d = be.reshape(-1)
    xs = _sc_scatter_rows(x_flat, pos1d)
    ys = _run_group_ffn(be1d, xs, W_up, W_down)
    yg = _sc_gather_rows(ys, pos1d)
    out = _run_combine(x_flat, yg, wgt, W_up_shared, W_down_shared)
    return out.reshape(bsz, seq, hidden), probs


# speedup vs baseline: 4.5255x; 1.0451x over previous
import functools

import jax
import jax.numpy as jnp
from jax import lax
from jax.experimental import pallas as pl
from jax.experimental.pallas import tpu as pltpu
from jax.experimental.pallas import tpu_sc as plsc

HIDDEN = 768
NUM_EXPERTS = 64
EXPERT_DIM = 512
SEQ = 2048

TILE_T = 64
TILES = SEQ // TILE_T + NUM_EXPERTS
ROWS = TILES * TILE_T

_INTERPRET = False


def _router_body(x_ref, wr_ref, probs_ref, pos_ref, wgt_ref, be_ref):
    x = x_ref[...]
    wr = wr_ref[...]
    logits = lax.dot_general(x, wr, (((1,), (1,)), ((), ())),
                             preferred_element_type=jnp.float32)
    m = jnp.max(logits, axis=1, keepdims=True)
    ex = jnp.exp(logits - m)
    probs = ex / jnp.sum(ex, axis=1, keepdims=True)
    probs_ref[...] = probs

    pmax = jnp.max(probs, axis=1, keepdims=True)
    col = lax.broadcasted_iota(jnp.int32, (SEQ, NUM_EXPERTS), 1)
    eidx = jnp.min(jnp.where(probs >= pmax, col, NUM_EXPERTS), axis=1,
                   keepdims=True)
    wgt_ref[...] = pmax / (pmax + 1e-9)

    onehot = (col == eidx).astype(jnp.float32)

    ones_row = jnp.ones((8, SEQ), dtype=jnp.float32)
    counts = lax.dot_general(ones_row, onehot, (((1,), (0,)), ((), ())),
                             preferred_element_type=jnp.float32)[0:1]
    ptc = jnp.floor((counts + (TILE_T - 1)) * (1.0 / TILE_T))
    er = lax.broadcasted_iota(jnp.int32, (NUM_EXPERTS, NUM_EXPERTS), 0)
    ec = lax.broadcasted_iota(jnp.int32, (NUM_EXPERTS, NUM_EXPERTS), 1)
    upper = (er <= ec).astype(jnp.float32)
    tile_start = lax.dot_general(ptc, upper, (((1,), (0,)), ((), ())),
                                 preferred_element_type=jnp.float32) - ptc
    row_start = tile_start * float(TILE_T)

    tr = lax.broadcasted_iota(jnp.int32, (SEQ, SEQ), 0)
    tc = lax.broadcasted_iota(jnp.int32, (SEQ, SEQ), 1)
    lower = (tc <= tr).astype(jnp.float32)
    csum = lax.dot_general(lower, onehot, (((1,), (0,)), ((), ())),
                           preferred_element_type=jnp.float32)
    posmat = onehot * (csum - 1.0 + row_start)
    pos = jnp.sum(posmat, axis=1, keepdims=True)
    pos_ref[...] = pos.astype(jnp.int32)

    ti = lax.broadcasted_iota(jnp.int32, (TILES, NUM_EXPERTS), 0).astype(
        jnp.float32)
    le = (jnp.broadcast_to(tile_start, (TILES, NUM_EXPERTS)) <= ti)
    be = jnp.sum(le.astype(jnp.float32), axis=1, keepdims=True) - 1.0
    be_ref[...] = be.astype(jnp.int32)


def _run_router(x_flat, W_router):
    return pl.pallas_call(
        _router_body,
        out_shape=(
            jax.ShapeDtypeStruct((SEQ, NUM_EXPERTS), jnp.float32),
            jax.ShapeDtypeStruct((SEQ, 1), jnp.int32),
            jax.ShapeDtypeStruct((SEQ, 1), jnp.float32),
            jax.ShapeDtypeStruct((TILES, 1), jnp.int32),
        ),
        interpret=_INTERPRET,
    )(x_flat, W_router)


def _group_ffn_body(be_ref, x_ref, wu_ref, wd_ref, o_ref):
    xb = x_ref[...]
    wu = wu_ref[0]
    wd = wd_ref[0]
    y = lax.dot_general(xb, wu, (((1,), (1,)), ((), ())),
                        preferred_element_type=jnp.float32)
    gate = y[:, :EXPERT_DIM]
    up = y[:, EXPERT_DIM:]
    h = gate * jax.nn.sigmoid(gate) * up
    o_ref[...] = lax.dot_general(h, wd, (((1,), (1,)), ((), ())),
                                 preferred_element_type=jnp.float32)


def _run_group_ffn(be, xs, W_up, W_down):
    grid_spec = pltpu.PrefetchScalarGridSpec(
        num_scalar_prefetch=1,
        grid=(TILES,),
        in_specs=[
            pl.BlockSpec((TILE_T, HIDDEN), lambda i, be_s: (i, 0)),
            pl.BlockSpec((1, 2 * EXPERT_DIM, HIDDEN),
                         lambda i, be_s: (be_s[i], 0, 0)),
            pl.BlockSpec((1, HIDDEN, EXPERT_DIM),
                         lambda i, be_s: (be_s[i], 0, 0)),
        ],
        out_specs=pl.BlockSpec((TILE_T, HIDDEN), lambda i, be_s: (i, 0)),
    )
    return pl.pallas_call(
        _group_ffn_body,
        grid_spec=grid_spec,
        out_shape=jax.ShapeDtypeStruct((ROWS, HIDDEN), jnp.float32),
        interpret=_INTERPRET,
    )(be, xs, W_up, W_down)


def _combine_body(x_ref, yg_ref, wgt_ref, wu_ref, wd_ref, o_ref):
    x = x_ref[...]
    wu = wu_ref[...]
    wd = wd_ref[...]
    y = lax.dot_general(x, wu, (((1,), (1,)), ((), ())),
                        preferred_element_type=jnp.float32)
    gate = y[:, :EXPERT_DIM]
    up = y[:, EXPERT_DIM:]
    h = gate * jax.nn.sigmoid(gate) * up
    shared = lax.dot_general(h, wd, (((1,), (1,)), ((), ())),
                             preferred_element_type=jnp.float32)
    o_ref[...] = shared + wgt_ref[...] * yg_ref[...]


def _run_combine(x_flat, yg, wgt, W_up_shared, W_down_shared):
    blk = 256
    grid_spec = pl.GridSpec(
        grid=(SEQ // blk,),
        in_specs=[
            pl.BlockSpec((blk, HIDDEN), lambda i: (i, 0)),
            pl.BlockSpec((blk, HIDDEN), lambda i: (i, 0)),
            pl.BlockSpec((blk, 1), lambda i: (i, 0)),
            pl.BlockSpec((2 * EXPERT_DIM, HIDDEN), lambda i: (0, 0)),
            pl.BlockSpec((HIDDEN, EXPERT_DIM), lambda i: (0, 0)),
        ],
        out_specs=pl.BlockSpec((blk, HIDDEN), lambda i: (i, 0)),
    )
    return pl.pallas_call(
        _combine_body,
        grid_spec=grid_spec,
        out_shape=jax.ShapeDtypeStruct((SEQ, HIDDEN), jnp.float32),
        interpret=_INTERPRET,
    )(x_flat, yg, wgt, W_up_shared, W_down_shared)


_SC_INFO = plsc.get_sparse_core_info()
_NW = _SC_INFO.num_cores * _SC_INFO.num_subcores
_TPW = SEQ // _NW


def _sc_mesh():
    return plsc.VectorSubcoreMesh(core_axis_name="c", subcore_axis_name="s")


def _sc_worker_base():
    wid = lax.axis_index("s") * _SC_INFO.num_cores + lax.axis_index("c")
    return wid * _TPW


def _sc_scatter_body(x_hbm, pos_hbm, xs_hbm, idx_v, rows_v, sem):
    base = _sc_worker_base()
    pltpu.sync_copy(pos_hbm.at[pl.ds(base, _TPW)], idx_v)
    pltpu.sync_copy(x_hbm.at[pl.ds(base, _TPW)], rows_v)
    pltpu.async_copy(rows_v, xs_hbm.at[idx_v], sem).wait()


def _sc_scatter_rows(x_flat, pos1d):
    return pl.kernel(
        _sc_scatter_body,
        mesh=_sc_mesh(),
        out_type=jax.ShapeDtypeStruct((ROWS, HIDDEN), jnp.float32),
        scratch_types=[
            pltpu.VMEM((_TPW,), jnp.int32),
            pltpu.VMEM((_TPW, HIDDEN), jnp.float32),
            pltpu.SemaphoreType.DMA,
        ],
    )(x_flat, pos1d)


def _sc_gather_body(ys_hbm, pos_hbm, yg_hbm, idx_v, rows_v, sem):
    base = _sc_worker_base()
    pltpu.sync_copy(pos_hbm.at[pl.ds(base, _TPW)], idx_v)
    pltpu.async_copy(ys_hbm.at[idx_v], rows_v, sem).wait()
    pltpu.sync_copy(rows_v, yg_hbm.at[pl.ds(base, _TPW)])


def _sc_gather_rows(ys, pos1d):
    return pl.kernel(
        _sc_gather_body,
        mesh=_sc_mesh(),
        out_type=jax.ShapeDtypeStruct((SEQ, HIDDEN), jnp.float32),
        scratch_types=[
            pltpu.VMEM((_TPW,), jnp.int32),
            pltpu.VMEM((_TPW, HIDDEN), jnp.float32),
            pltpu.SemaphoreType.DMA,
        ],
    )(ys, pos1d)


def kernel(x, W_router, W_up, W_down, W_up_shared, W_down_shared):
    bsz, seq, hidden = x.shape
    x_flat = x.reshape(-1, hidden)
    probs, pos, wgt, be = _run_router(x_flat, W_router)
    pos1d = pos.reshape(-1)
    be1d = be.reshape(-1)
    xs = _sc_scatter_rows(x_flat, pos1d)
    ys = _run_group_ffn(be1d, xs, W_up, W_down)
    yg = _sc_gather_rows(ys, pos1d)
    out = _run_combine(x_flat, yg, wgt, W_up_shared, W_down_shared)
    return out.reshape(bsz, seq, hidden), probs


# speedup vs baseline: 5.3114x; 1.1736x over previous
import functools

import jax
import jax.numpy as jnp
from jax import lax
from jax.experimental import pallas as pl
from jax.experimental.pallas import tpu as pltpu
from jax.experimental.pallas import tpu_sc as plsc

HIDDEN = 768
NUM_EXPERTS = 64
EXPERT_DIM = 512
SEQ = 2048

TILE_T = 64
TILES = SEQ // TILE_T + NUM_EXPERTS
ROWS = TILES * TILE_T

_INTERPRET = False


def _router_body(x_ref, wr_ref, probs_ref, pos_ref, wgt_ref, be_ref):
    x = x_ref[...]
    wr = wr_ref[...]
    logits = lax.dot_general(x, wr, (((1,), (1,)), ((), ())),
                             preferred_element_type=jnp.float32)
    m = jnp.max(logits, axis=1, keepdims=True)
    ex = jnp.exp(logits - m)
    probs = ex / jnp.sum(ex, axis=1, keepdims=True)
    probs_ref[...] = probs

    pmax = jnp.max(probs, axis=1, keepdims=True)
    col = lax.broadcasted_iota(jnp.int32, (SEQ, NUM_EXPERTS), 1)
    eidx = jnp.min(jnp.where(probs >= pmax, col, NUM_EXPERTS), axis=1,
                   keepdims=True)
    wgt_ref[...] = pmax / (pmax + 1e-9)

    onehot = (col == eidx).astype(jnp.float32)

    ones_row = jnp.ones((8, SEQ), dtype=jnp.float32)
    counts = lax.dot_general(ones_row, onehot, (((1,), (0,)), ((), ())),
                             preferred_element_type=jnp.float32)[0:1]
    ptc = jnp.floor((counts + (TILE_T - 1)) * (1.0 / TILE_T))
    er = lax.broadcasted_iota(jnp.int32, (NUM_EXPERTS, NUM_EXPERTS), 0)
    ec = lax.broadcasted_iota(jnp.int32, (NUM_EXPERTS, NUM_EXPERTS), 1)
    upper = (er <= ec).astype(jnp.float32)
    tile_start = lax.dot_general(ptc, upper, (((1,), (0,)), ((), ())),
                                 preferred_element_type=jnp.float32) - ptc
    row_start = tile_start * float(TILE_T)

    tr = lax.broadcasted_iota(jnp.int32, (SEQ, SEQ), 0)
    tc = lax.broadcasted_iota(jnp.int32, (SEQ, SEQ), 1)
    lower = (tc <= tr).astype(jnp.float32)
    csum = lax.dot_general(lower, onehot, (((1,), (0,)), ((), ())),
                           preferred_element_type=jnp.float32)
    posmat = onehot * (csum - 1.0 + row_start)
    pos = jnp.sum(posmat, axis=1, keepdims=True)
    pos_ref[...] = pos.astype(jnp.int32)

    ti = lax.broadcasted_iota(jnp.int32, (TILES + 1, NUM_EXPERTS), 0).astype(
        jnp.float32)
    le = (jnp.broadcast_to(tile_start, (TILES + 1, NUM_EXPERTS)) <= ti)
    be = jnp.sum(le.astype(jnp.float32), axis=1, keepdims=True) - 1.0
    ntiles = jnp.sum(ptc, axis=1, keepdims=True)
    riota = lax.broadcasted_iota(jnp.int32, (TILES + 1, 1), 0)
    be = jnp.where(riota == TILES, jnp.broadcast_to(ntiles, (TILES + 1, 1)), be)
    be_ref[...] = be.astype(jnp.int32)


def _run_router(x_flat, W_router):
    return pl.pallas_call(
        _router_body,
        out_shape=(
            jax.ShapeDtypeStruct((SEQ, NUM_EXPERTS), jnp.float32),
            jax.ShapeDtypeStruct((SEQ, 1), jnp.int32),
            jax.ShapeDtypeStruct((SEQ, 1), jnp.float32),
            jax.ShapeDtypeStruct((TILES + 1, 1), jnp.int32),
        ),
        interpret=_INTERPRET,
    )(x_flat, W_router)


def _group_ffn_body(be_ref, x_ref, wu_ref, wd_ref, o_ref):
    @pl.when(pl.program_id(0) < be_ref[TILES])
    def _():
        xb = x_ref[...]
        wu = wu_ref[0]
        wd = wd_ref[0]
        y = lax.dot_general(xb, wu, (((1,), (1,)), ((), ())),
                            preferred_element_type=jnp.float32)
        gate = y[:, :EXPERT_DIM]
        up = y[:, EXPERT_DIM:]
        h = gate * jax.nn.sigmoid(gate) * up
        o_ref[...] = lax.dot_general(h, wd, (((1,), (1,)), ((), ())),
                                     preferred_element_type=jnp.float32)


def _run_group_ffn(be, xs, W_up, W_down):
    def live(i, be_s):
        return jnp.minimum(i, be_s[TILES] - 1)

    grid_spec = pltpu.PrefetchScalarGridSpec(
        num_scalar_prefetch=1,
        grid=(TILES,),
        in_specs=[
            pl.BlockSpec((TILE_T, HIDDEN), lambda i, be_s: (live(i, be_s), 0)),
            pl.BlockSpec((1, 2 * EXPERT_DIM, HIDDEN),
                         lambda i, be_s: (be_s[live(i, be_s)], 0, 0)),
            pl.BlockSpec((1, HIDDEN, EXPERT_DIM),
                         lambda i, be_s: (be_s[live(i, be_s)], 0, 0)),
        ],
        out_specs=pl.BlockSpec((TILE_T, HIDDEN),
                               lambda i, be_s: (live(i, be_s), 0)),
    )
    return pl.pallas_call(
        _group_ffn_body,
        grid_spec=grid_spec,
        out_shape=jax.ShapeDtypeStruct((ROWS, HIDDEN), jnp.float32),
        interpret=_INTERPRET,
    )(be, xs, W_up, W_down)


def _combine_body(x_ref, yg_ref, wgt_ref, wu_ref, wd_ref, o_ref):
    x = x_ref[...]
    wu = wu_ref[...]
    wd = wd_ref[...]
    y = lax.dot_general(x, wu, (((1,), (1,)), ((), ())),
                        preferred_element_type=jnp.float32)
    gate = y[:, :EXPERT_DIM]
    up = y[:, EXPERT_DIM:]
    h = gate * jax.nn.sigmoid(gate) * up
    shared = lax.dot_general(h, wd, (((1,), (1,)), ((), ())),
                             preferred_element_type=jnp.float32)
    o_ref[...] = shared + wgt_ref[...] * yg_ref[...]


def _run_combine(x_flat, yg, wgt, W_up_shared, W_down_shared):
    blk = 256
    grid_spec = pl.GridSpec(
        grid=(SEQ // blk,),
        in_specs=[
            pl.BlockSpec((blk, HIDDEN), lambda i: (i, 0)),
            pl.BlockSpec((blk, HIDDEN), lambda i: (i, 0)),
            pl.BlockSpec((blk, 1), lambda i: (i, 0)),
            pl.BlockSpec((2 * EXPERT_DIM, HIDDEN), lambda i: (0, 0)),
            pl.BlockSpec((HIDDEN, EXPERT_DIM), lambda i: (0, 0)),
        ],
        out_specs=pl.BlockSpec((blk, HIDDEN), lambda i: (i, 0)),
    )
    return pl.pallas_call(
        _combine_body,
        grid_spec=grid_spec,
        out_shape=jax.ShapeDtypeStruct((SEQ, HIDDEN), jnp.float32),
        interpret=_INTERPRET,
    )(x_flat, yg, wgt, W_up_shared, W_down_shared)


_SC_NC = 2
_SC_NS = 16
_NW = _SC_NC * _SC_NS
_TPW = SEQ // _NW


def _sc_mesh():
    return plsc.VectorSubcoreMesh(core_axis_name="c", subcore_axis_name="s")


def _sc_worker_base():
    wid = lax.axis_index("s") * _SC_NC + lax.axis_index("c")
    return wid * _TPW


def _sc_scatter_body(x_hbm, pos_hbm, xs_hbm, idx_v, rows_v, sem):
    base = _sc_worker_base()
    pltpu.sync_copy(pos_hbm.at[pl.ds(base, _TPW)], idx_v)
    pltpu.sync_copy(x_hbm.at[pl.ds(base, _TPW)], rows_v)
    pltpu.async_copy(rows_v, xs_hbm.at[idx_v], sem).wait()


def _sc_scatter_rows(x_flat, pos1d):
    return pl.kernel(
        _sc_scatter_body,
        mesh=_sc_mesh(),
        out_type=jax.ShapeDtypeStruct((ROWS, HIDDEN), jnp.float32),
        scratch_types=[
            pltpu.VMEM((_TPW,), jnp.int32),
            pltpu.VMEM((_TPW, HIDDEN), jnp.float32),
            pltpu.SemaphoreType.DMA,
        ],
    )(x_flat, pos1d)


def _sc_gather_body(ys_hbm, pos_hbm, yg_hbm, idx_v, rows_v, sem):
    base = _sc_worker_base()
    pltpu.sync_copy(pos_hbm.at[pl.ds(base, _TPW)], idx_v)
    pltpu.async_copy(ys_hbm.at[idx_v], rows_v, sem).wait()
    pltpu.sync_copy(rows_v, yg_hbm.at[pl.ds(base, _TPW)])


def _sc_gather_rows(ys, pos1d):
    return pl.kernel(
        _sc_gather_body,
        mesh=_sc_mesh(),
        out_type=jax.ShapeDtypeStruct((SEQ, HIDDEN), jnp.float32),
        scratch_types=[
            pltpu.VMEM((_TPW,), jnp.int32),
            pltpu.VMEM((_TPW, HIDDEN), jnp.float32),
            pltpu.SemaphoreType.DMA,
        ],
    )(ys, pos1d)


def kernel(x, W_router, W_up, W_down, W_up_shared, W_down_shared):
    bsz, seq, hidden = x.shape
    x_flat = x.reshape(-1, hidden)
    probs, pos, wgt, be = _run_router(x_flat, W_router)
    pos1d = pos.reshape(-1)
    be1d = be.reshape(-1)
    xs = _sc_scatter_rows(x_flat, pos1d)
    ys = _run_group_ffn(be1d, xs, W_up, W_down)
    yg = _sc_gather_rows(ys, pos1d)
    out = _run_combine(x_flat, yg, wgt, W_up_shared, W_down_shared)
    return out.reshape(bsz, seq, hidden), probs


# speedup vs baseline: 5.4164x; 1.0198x over previous
import functools

import jax
import jax.numpy as jnp
from jax import lax
from jax.experimental import pallas as pl
from jax.experimental.pallas import tpu as pltpu
from jax.experimental.pallas import tpu_sc as plsc

HIDDEN = 768
NUM_EXPERTS = 64
EXPERT_DIM = 512
SEQ = 2048

TILE_T = 64
TILES = SEQ // TILE_T + NUM_EXPERTS
ROWS = TILES * TILE_T

_INTERPRET = False


def _router_body(x_ref, wr_ref, probs_ref, pos_ref, wgt_ref, be_ref):
    x = x_ref[...]
    wr = wr_ref[...]
    logits = lax.dot_general(x, wr, (((1,), (1,)), ((), ())),
                             preferred_element_type=jnp.float32)
    m = jnp.max(logits, axis=1, keepdims=True)
    ex = jnp.exp(logits - m)
    probs = ex / jnp.sum(ex, axis=1, keepdims=True)
    probs_ref[...] = probs

    pmax = jnp.max(probs, axis=1, keepdims=True)
    col = lax.broadcasted_iota(jnp.int32, (SEQ, NUM_EXPERTS), 1)
    eidx = jnp.min(jnp.where(probs >= pmax, col, NUM_EXPERTS), axis=1,
                   keepdims=True)
    wgt_ref[...] = pmax / (pmax + 1e-9)

    onehot = (col == eidx).astype(jnp.float32)

    ones_row = jnp.ones((8, SEQ), dtype=jnp.float32)
    counts = lax.dot_general(ones_row, onehot, (((1,), (0,)), ((), ())),
                             preferred_element_type=jnp.float32)[0:1]
    ptc = jnp.floor((counts + (TILE_T - 1)) * (1.0 / TILE_T))
    er = lax.broadcasted_iota(jnp.int32, (NUM_EXPERTS, NUM_EXPERTS), 0)
    ec = lax.broadcasted_iota(jnp.int32, (NUM_EXPERTS, NUM_EXPERTS), 1)
    upper = (er <= ec).astype(jnp.float32)
    tile_start = lax.dot_general(ptc, upper, (((1,), (0,)), ((), ())),
                                 preferred_element_type=jnp.float32) - ptc
    row_start = tile_start * float(TILE_T)

    chunk = 128
    cr = lax.broadcasted_iota(jnp.int32, (chunk, chunk), 0)
    cc = lax.broadcasted_iota(jnp.int32, (chunk, chunk), 1)
    l128 = (cc <= cr).astype(jnp.float32)
    running = jnp.zeros((1, NUM_EXPERTS), jnp.float32)
    for k in range(SEQ // chunk):
        oh_k = onehot[k * chunk:(k + 1) * chunk]
        c_k = lax.dot_general(l128, oh_k, (((1,), (0,)), ((), ())),
                              preferred_element_type=jnp.float32) + running
        running = c_k[chunk - 1:chunk, :]
        posmat = oh_k * (c_k - 1.0 + row_start)
        pos_ref[k * chunk:(k + 1) * chunk, :] = jnp.sum(
            posmat, axis=1, keepdims=True).astype(jnp.int32)

    ti = lax.broadcasted_iota(jnp.int32, (TILES + 1, NUM_EXPERTS), 0).astype(
        jnp.float32)
    le = (jnp.broadcast_to(tile_start, (TILES + 1, NUM_EXPERTS)) <= ti)
    be = jnp.sum(le.astype(jnp.float32), axis=1, keepdims=True) - 1.0
    ntiles = jnp.sum(ptc, axis=1, keepdims=True)
    riota = lax.broadcasted_iota(jnp.int32, (TILES + 1, 1), 0)
    be = jnp.where(riota == TILES, jnp.broadcast_to(ntiles, (TILES + 1, 1)), be)
    be_ref[...] = be.astype(jnp.int32)


def _run_router(x_flat, W_router):
    return pl.pallas_call(
        _router_body,
        out_shape=(
            jax.ShapeDtypeStruct((SEQ, NUM_EXPERTS), jnp.float32),
            jax.ShapeDtypeStruct((SEQ, 1), jnp.int32),
            jax.ShapeDtypeStruct((SEQ, 1), jnp.float32),
            jax.ShapeDtypeStruct((TILES + 1, 1), jnp.int32),
        ),
        interpret=_INTERPRET,
    )(x_flat, W_router)


def _group_ffn_body(be_ref, x_ref, wu_ref, wd_ref, o_ref):
    @pl.when(pl.program_id(0) < be_ref[TILES])
    def _():
        xb = x_ref[...]
        wu = wu_ref[0]
        wd = wd_ref[0]
        y = lax.dot_general(xb, wu, (((1,), (1,)), ((), ())),
                            preferred_element_type=jnp.float32)
        gate = y[:, :EXPERT_DIM]
        up = y[:, EXPERT_DIM:]
        h = gate * jax.nn.sigmoid(gate) * up
        o_ref[...] = lax.dot_general(h, wd, (((1,), (1,)), ((), ())),
                                     preferred_element_type=jnp.float32)


def _run_group_ffn(be, xs, W_up, W_down):
    def live(i, be_s):
        return jnp.minimum(i, be_s[TILES] - 1)

    grid_spec = pltpu.PrefetchScalarGridSpec(
        num_scalar_prefetch=1,
        grid=(TILES,),
        in_specs=[
            pl.BlockSpec((TILE_T, HIDDEN), lambda i, be_s: (live(i, be_s), 0)),
            pl.BlockSpec((1, 2 * EXPERT_DIM, HIDDEN),
                         lambda i, be_s: (be_s[live(i, be_s)], 0, 0)),
            pl.BlockSpec((1, HIDDEN, EXPERT_DIM),
                         lambda i, be_s: (be_s[live(i, be_s)], 0, 0)),
        ],
        out_specs=pl.BlockSpec((TILE_T, HIDDEN),
                               lambda i, be_s: (live(i, be_s), 0)),
    )
    return pl.pallas_call(
        _group_ffn_body,
        grid_spec=grid_spec,
        out_shape=jax.ShapeDtypeStruct((ROWS, HIDDEN), jnp.float32),
        interpret=_INTERPRET,
    )(be, xs, W_up, W_down)


def _combine_body(x_ref, yg_ref, wgt_ref, wu_ref, wd_ref, o_ref):
    x = x_ref[...]
    wu = wu_ref[...]
    wd = wd_ref[...]
    y = lax.dot_general(x, wu, (((1,), (1,)), ((), ())),
                        preferred_element_type=jnp.float32)
    gate = y[:, :EXPERT_DIM]
    up = y[:, EXPERT_DIM:]
    h = gate * jax.nn.sigmoid(gate) * up
    shared = lax.dot_general(h, wd, (((1,), (1,)), ((), ())),
                             preferred_element_type=jnp.float32)
    o_ref[...] = shared + wgt_ref[...] * yg_ref[...]


def _run_combine(x_flat, yg, wgt, W_up_shared, W_down_shared):
    blk = 256
    grid_spec = pl.GridSpec(
        grid=(SEQ // blk,),
        in_specs=[
            pl.BlockSpec((blk, HIDDEN), lambda i: (i, 0)),
            pl.BlockSpec((blk, HIDDEN), lambda i: (i, 0)),
            pl.BlockSpec((blk, 1), lambda i: (i, 0)),
            pl.BlockSpec((2 * EXPERT_DIM, HIDDEN), lambda i: (0, 0)),
            pl.BlockSpec((HIDDEN, EXPERT_DIM), lambda i: (0, 0)),
        ],
        out_specs=pl.BlockSpec((blk, HIDDEN), lambda i: (i, 0)),
    )
    return pl.pallas_call(
        _combine_body,
        grid_spec=grid_spec,
        out_shape=jax.ShapeDtypeStruct((SEQ, HIDDEN), jnp.float32),
        interpret=_INTERPRET,
    )(x_flat, yg, wgt, W_up_shared, W_down_shared)


_SC_NC = 2
_SC_NS = 16
_NW = _SC_NC * _SC_NS
_TPW = SEQ // _NW


def _sc_mesh():
    return plsc.VectorSubcoreMesh(core_axis_name="c", subcore_axis_name="s")


def _sc_worker_base():
    wid = lax.axis_index("s") * _SC_NC + lax.axis_index("c")
    return wid * _TPW


def _sc_scatter_body(x_hbm, pos_hbm, xs_hbm, idx_v, rows_v, sem):
    base = _sc_worker_base()
    pltpu.sync_copy(pos_hbm.at[pl.ds(base, _TPW)], idx_v)
    pltpu.sync_copy(x_hbm.at[pl.ds(base, _TPW)], rows_v)
    pltpu.async_copy(rows_v, xs_hbm.at[idx_v], sem).wait()


def _sc_scatter_rows(x_flat, pos1d):
    return pl.kernel(
        _sc_scatter_body,
        mesh=_sc_mesh(),
        out_type=jax.ShapeDtypeStruct((ROWS, HIDDEN), jnp.float32),
        scratch_types=[
            pltpu.VMEM((_TPW,), jnp.int32),
            pltpu.VMEM((_TPW, HIDDEN), jnp.float32),
            pltpu.SemaphoreType.DMA,
        ],
    )(x_flat, pos1d)


def _sc_gather_body(ys_hbm, pos_hbm, yg_hbm, idx_v, rows_v, sem):
    base = _sc_worker_base()
    pltpu.sync_copy(pos_hbm.at[pl.ds(base, _TPW)], idx_v)
    pltpu.async_copy(ys_hbm.at[idx_v], rows_v, sem).wait()
    pltpu.sync_copy(rows_v, yg_hbm.at[pl.ds(base, _TPW)])


def _sc_gather_rows(ys, pos1d):
    return pl.kernel(
        _sc_gather_body,
        mesh=_sc_mesh(),
        out_type=jax.ShapeDtypeStruct((SEQ, HIDDEN), jnp.float32),
        scratch_types=[
            pltpu.VMEM((_TPW,), jnp.int32),
            pltpu.VMEM((_TPW, HIDDEN), jnp.float32),
            pltpu.SemaphoreType.DMA,
        ],
    )(ys, pos1d)


def kernel(x, W_router, W_up, W_down, W_up_shared, W_down_shared):
    bsz, seq, hidden = x.shape
    x_flat = x.reshape(-1, hidden)
    probs, pos, wgt, be = _run_router(x_flat, W_router)
    pos1d = pos.reshape(-1)
    be1d = be.reshape(-1)
    xs = _sc_scatter_rows(x_flat, pos1d)
    ys = _run_group_ffn(be1d, xs, W_up, W_down)
    yg = _sc_gather_rows(ys, pos1d)
    out = _run_combine(x_flat, yg, wgt, W_up_shared, W_down_shared)
    return out.reshape(bsz, seq, hidden), probs
